# baseline (speedup 1.0000x reference)
import functools

import jax
import jax.numpy as jnp
import numpy as np
from jax.experimental import pallas as pl
from jax.experimental.pallas import tpu as pltpu

_LIPS = np.array([61, 146, 91, 181, 84, 17, 314, 405, 321, 375,
                  78, 191, 80, 81, 82, 13, 312, 311, 310, 415,
                  95, 88, 178, 87, 14, 317, 402, 318, 324, 308,
                  291, 185, 40, 39, 37, 0, 267, 269, 270, 409], dtype=np.int64)

_NFRAMES = 16384
_NLM = 543
_NOUT = 164
_T_BLK = 2048
_GRID_T = _NFRAMES // _T_BLK

_IDX82 = np.concatenate([np.arange(468, 489), np.arange(522, 543), _LIPS])

_GROUPS = sorted({int(l) // 8 for l in _IDX82})
_NG = len(_GROUPS)
_NSTEPS = 2 * _NG
_KROWS = 2 * 8 * _NG


def _build_tables():
    ginv = {g: gi for gi, g in enumerate(_GROUPS)}
    gmat = np.zeros((_KROWS, _NOUT), np.float32)
    for k, l in enumerate(_IDX82):
        l = int(l)
        for c in (0, 1):
            gmat[c * 8 * _NG + ginv[l // 8] * 8 + l % 8, 2 * k + c] = 1.0
    c_of = np.arange(_NSTEPS, dtype=np.int32) // _NG
    g_of = np.array([_GROUPS[j % _NG] for j in range(_NSTEPS)], np.int32)
    return gmat, c_of, g_of


_GMAT, _C_OF, _G_OF = _build_tables()


def _gather_body(c_ref, g_of_ref, ft_ref, g_ref, out_ref, scratch_ref):
    j = pl.program_id(1)
    x = ft_ref[0]
    x = jnp.where(jnp.isnan(x), 0.0, x)
    scratch_ref[pl.ds(j * 8, 8), :] = x

    @pl.when(j == _NSTEPS - 1)
    def _():
        dn = (((0,), (0,)), ((), ()))
        out_ref[...] = jax.lax.dot_general(
            g_ref[...], scratch_ref[...], dn,
            precision=jax.lax.Precision.DEFAULT,
            preferred_element_type=jnp.float32)


@functools.cache
def _make_tc_gather():
    return pl.pallas_call(
        _gather_body,
        grid_spec=pltpu.PrefetchScalarGridSpec(
            num_scalar_prefetch=2,
            grid=(_GRID_T, _NSTEPS),
            in_specs=[
                pl.BlockSpec((1, 8, _T_BLK),
                             lambda i, j, c_of, g_of: (c_of[j], g_of[j], i)),
                pl.BlockSpec((_KROWS, _NOUT), lambda i, j, c_of, g_of: (0, 0)),
            ],
            out_specs=pl.BlockSpec((_NOUT, _T_BLK),
                                   lambda i, j, c_of, g_of: (0, i)),
            scratch_shapes=[pltpu.VMEM((_KROWS, _T_BLK), jnp.float32)],
        ),
        out_shape=jax.ShapeDtypeStruct((_NOUT, _NFRAMES), jnp.float32),
        compiler_params=pltpu.CompilerParams(
            dimension_semantics=("arbitrary", "arbitrary"),
        ),
    )


def kernel(frames):
    ft = frames.transpose(2, 1, 0)
    out = _make_tc_gather()(
        jnp.asarray(_C_OF), jnp.asarray(_G_OF), ft, jnp.asarray(_GMAT))
    return out.T


# speedup vs baseline: 11.0662x; 11.0662x over previous
import functools

import jax
import jax.numpy as jnp
import numpy as np
from jax.experimental import pallas as pl
from jax.experimental.pallas import tpu as pltpu

_LIPS = np.array([61, 146, 91, 181, 84, 17, 314, 405, 321, 375,
                  78, 191, 80, 81, 82, 13, 312, 311, 310, 415,
                  95, 88, 178, 87, 14, 317, 402, 318, 324, 308,
                  291, 185, 40, 39, 37, 0, 267, 269, 270, 409], dtype=np.int64)

_NFRAMES = 16384
_NLM = 543
_NOUT = 164
_T_BLK = 2048
_GRID_T = _NFRAMES // _T_BLK

_IDX82 = np.concatenate([np.arange(468, 489), np.arange(522, 543), _LIPS])

_GROUPS = sorted({int(l) // 8 for l in _IDX82})
_NG = len(_GROUPS)
_CG = [(c, g) for c in (0, 1) for g in _GROUPS]
_KROWS = 8 * len(_CG)


def _build_gmat() -> np.ndarray:
    ginv = {g: gi for gi, g in enumerate(_GROUPS)}
    gmat = np.zeros((_KROWS, _NOUT), np.float32)
    for k, l in enumerate(_IDX82):
        l = int(l)
        for c in (0, 1):
            gmat[(c * _NG + ginv[l // 8]) * 8 + l % 8, 2 * k + c] = 1.0
    return gmat


_GMAT = _build_gmat()


def _gather_body(ft_hbm, g_ref, out_ref, scratch_ref, sem_ref):
    i = pl.program_id(0)

    def copies(slot, chunk):
        return [
            pltpu.make_async_copy(
                ft_hbm.at[c, pl.ds(8 * g, min(8, _NLM - 8 * g)),
                          pl.ds(chunk * _T_BLK, _T_BLK)],
                scratch_ref.at[slot, pl.ds(8 * s, min(8, _NLM - 8 * g)), :],
                sem_ref.at[slot],
            )
            for s, (c, g) in enumerate(_CG)
        ]

    @pl.when(i == 0)
    def _():
        for cp in copies(0, 0):
            cp.start()

    @pl.when(i + 1 < _GRID_T)
    def _():
        nxt = i + 1
        for cp in copies((i + 1) % 2, nxt):
            cp.start()

    slot = i % 2
    for cp in copies(slot, i):
        cp.wait()

    x = scratch_ref[slot]
    x = jnp.where(jnp.isnan(x), 0.0, x)
    dn = (((0,), (0,)), ((), ()))
    out_ref[...] = jax.lax.dot_general(
        g_ref[...], x, dn,
        precision=jax.lax.Precision.DEFAULT,
        preferred_element_type=jnp.float32)


@functools.cache
def _make_tc_gather():
    return pl.pallas_call(
        _gather_body,
        grid=(_GRID_T,),
        in_specs=[
            pl.BlockSpec(memory_space=pl.ANY),
            pl.BlockSpec((_KROWS, _NOUT), lambda i: (0, 0)),
        ],
        out_specs=pl.BlockSpec((_NOUT, _T_BLK), lambda i: (0, i)),
        out_shape=jax.ShapeDtypeStruct((_NOUT, _NFRAMES), jnp.float32),
        scratch_shapes=[
            pltpu.VMEM((2, _KROWS, _T_BLK), jnp.float32),
            pltpu.SemaphoreType.DMA((2,)),
        ],
        compiler_params=pltpu.CompilerParams(
            dimension_semantics=("arbitrary",),
        ),
    )


def kernel(frames):
    ft = frames.transpose(2, 1, 0)
    out = _make_tc_gather()(ft, jnp.asarray(_GMAT))
    return out.T


# speedup vs baseline: 18.3140x; 1.6550x over previous
import functools

import jax
import jax.numpy as jnp
import numpy as np
from jax.experimental import pallas as pl
from jax.experimental.pallas import tpu as pltpu

_LIPS = np.array([61, 146, 91, 181, 84, 17, 314, 405, 321, 375,
                  78, 191, 80, 81, 82, 13, 312, 311, 310, 415,
                  95, 88, 178, 87, 14, 317, 402, 318, 324, 308,
                  291, 185, 40, 39, 37, 0, 267, 269, 270, 409], dtype=np.int64)

_NFRAMES = 16384
_NLM = 543
_NOUT = 164
_T_BLK = 2048
_GRID_T = _NFRAMES // _T_BLK

_IDX82 = np.concatenate([np.arange(468, 489), np.arange(522, 543), _LIPS])
_ROWS = [(m % 2, int(_IDX82[m // 2])) for m in range(_NOUT)]


def _gather_body(ft_hbm, out_ref, scratch_ref, sem_ref):
    i = pl.program_id(0)

    def copies(slot, chunk):
        return [
            pltpu.make_async_copy(
                ft_hbm.at[c, pl.ds(l, 1), pl.ds(chunk * _T_BLK, _T_BLK)],
                scratch_ref.at[slot, pl.ds(m, 1), :],
                sem_ref.at[slot],
            )
            for m, (c, l) in enumerate(_ROWS)
        ]

    @pl.when(i == 0)
    def _():
        for cp in copies(0, 0):
            cp.start()

    @pl.when(i + 1 < _GRID_T)
    def _():
        for cp in copies((i + 1) % 2, i + 1):
            cp.start()

    slot = i % 2
    for cp in copies(slot, i):
        cp.wait()

    x = scratch_ref[slot]
    out_ref[...] = jnp.where(jnp.isnan(x), 0.0, x)


@functools.cache
def _make_tc_gather():
    return pl.pallas_call(
        _gather_body,
        grid=(_GRID_T,),
        in_specs=[pl.BlockSpec(memory_space=pl.ANY)],
        out_specs=pl.BlockSpec((_NOUT, _T_BLK), lambda i: (0, i)),
        out_shape=jax.ShapeDtypeStruct((_NOUT, _NFRAMES), jnp.float32),
        scratch_shapes=[
            pltpu.VMEM((2, _NOUT, _T_BLK), jnp.float32),
            pltpu.SemaphoreType.DMA((2,)),
        ],
        compiler_params=pltpu.CompilerParams(
            dimension_semantics=("arbitrary",),
        ),
    )


def kernel(frames):
    ft = frames.transpose(2, 1, 0)
    out = _make_tc_gather()(ft)
    return out.T


# speedup vs baseline: 22.1337x; 1.2086x over previous
import functools

import jax
import jax.numpy as jnp
import numpy as np
from jax.experimental import pallas as pl
from jax.experimental.pallas import tpu as pltpu

_LIPS = np.array([61, 146, 91, 181, 84, 17, 314, 405, 321, 375,
                  78, 191, 80, 81, 82, 13, 312, 311, 310, 415,
                  95, 88, 178, 87, 14, 317, 402, 318, 324, 308,
                  291, 185, 40, 39, 37, 0, 267, 269, 270, 409], dtype=np.int64)

_NFRAMES = 16384
_NLM = 543
_NOUT = 164
_T_BLK = 4096
_GRID_T = _NFRAMES // _T_BLK

_IDX82 = np.concatenate([np.arange(468, 489), np.arange(522, 543), _LIPS])
_ROWS = [(m % 2, int(_IDX82[m // 2])) for m in range(_NOUT)]


def _gather_body(ft_hbm, out_ref, scratch_ref, sem_ref):
    i = pl.program_id(0)

    def copies(slot, chunk):
        return [
            pltpu.make_async_copy(
                ft_hbm.at[c, pl.ds(l, 1), pl.ds(chunk * _T_BLK, _T_BLK)],
                scratch_ref.at[slot, pl.ds(m, 1), :],
                sem_ref.at[slot],
            )
            for m, (c, l) in enumerate(_ROWS)
        ]

    @pl.when(i == 0)
    def _():
        for cp in copies(0, 0):
            cp.start()

    @pl.when(i + 1 < _GRID_T)
    def _():
        for cp in copies((i + 1) % 2, i + 1):
            cp.start()

    slot = i % 2
    for cp in copies(slot, i):
        cp.wait()

    x = scratch_ref[slot]
    out_ref[...] = jnp.where(jnp.isnan(x), 0.0, x)


@functools.cache
def _make_tc_gather():
    return pl.pallas_call(
        _gather_body,
        grid=(_GRID_T,),
        in_specs=[pl.BlockSpec(memory_space=pl.ANY)],
        out_specs=pl.BlockSpec((_NOUT, _T_BLK), lambda i: (0, i)),
        out_shape=jax.ShapeDtypeStruct((_NOUT, _NFRAMES), jnp.float32),
        scratch_shapes=[
            pltpu.VMEM((2, _NOUT, _T_BLK), jnp.float32),
            pltpu.SemaphoreType.DMA((2,)),
        ],
        compiler_params=pltpu.CompilerParams(
            dimension_semantics=("arbitrary",),
        ),
    )


def kernel(frames):
    ft = frames.transpose(2, 1, 0)
    out = _make_tc_gather()(ft)
    return out.T


# speedup vs baseline: 26.2822x; 1.1874x over previous
import functools

import jax
import jax.numpy as jnp
import numpy as np
from jax.experimental import pallas as pl
from jax.experimental.pallas import tpu as pltpu

_LIPS = np.array([61, 146, 91, 181, 84, 17, 314, 405, 321, 375,
                  78, 191, 80, 81, 82, 13, 312, 311, 310, 415,
                  95, 88, 178, 87, 14, 317, 402, 318, 324, 308,
                  291, 185, 40, 39, 37, 0, 267, 269, 270, 409], dtype=np.int64)

_NFRAMES = 16384
_NLM = 543
_NOUT = 164
_T_BLK = 8192
_GRID_T = _NFRAMES // _T_BLK

_IDX82 = np.concatenate([np.arange(468, 489), np.arange(522, 543), _LIPS])
_ROWS = [(m % 2, int(_IDX82[m // 2])) for m in range(_NOUT)]


def _gather_body(ft_hbm, out_ref, scratch_ref, sem_ref):
    i = pl.program_id(0)

    def copies(slot, chunk):
        return [
            pltpu.make_async_copy(
                ft_hbm.at[c, pl.ds(l, 1), pl.ds(chunk * _T_BLK, _T_BLK)],
                scratch_ref.at[slot, pl.ds(m, 1), :],
                sem_ref.at[slot],
            )
            for m, (c, l) in enumerate(_ROWS)
        ]

    @pl.when(i == 0)
    def _():
        for cp in copies(0, 0):
            cp.start()

    @pl.when(i + 1 < _GRID_T)
    def _():
        for cp in copies((i + 1) % 2, i + 1):
            cp.start()

    slot = i % 2
    for cp in copies(slot, i):
        cp.wait()

    x = scratch_ref[slot]
    out_ref[...] = jnp.where(jnp.isnan(x), 0.0, x)


@functools.cache
def _make_tc_gather():
    return pl.pallas_call(
        _gather_body,
        grid=(_GRID_T,),
        in_specs=[pl.BlockSpec(memory_space=pl.ANY)],
        out_specs=pl.BlockSpec((_NOUT, _T_BLK), lambda i: (0, i)),
        out_shape=jax.ShapeDtypeStruct((_NOUT, _NFRAMES), jnp.float32),
        scratch_shapes=[
            pltpu.VMEM((2, _NOUT, _T_BLK), jnp.float32),
            pltpu.SemaphoreType.DMA((2,)),
        ],
        compiler_params=pltpu.CompilerParams(
            dimension_semantics=("arbitrary",),
        ),
    )


def kernel(frames):
    ft = frames.transpose(2, 1, 0)
    out = _make_tc_gather()(ft)
    return out.T
